# baseline (speedup 1.0000x reference)
import functools

import jax
import jax.numpy as jnp
from jax import lax
from jax.experimental import pallas as pl
from jax.experimental.pallas import tpu as pltpu
from jax.experimental.pallas import tpu_sc as plsc

HIDDEN = 128
NC = 2
NS = 16
NW = NC * NS
CH = 128
NBUF = 4
LOOK = 2


@functools.lru_cache(maxsize=None)
def _make_emb(B):
    assert B % (NW * CH) == 0
    bpw = B // NW
    nch = bpw // CH
    assert nch % NBUF == 2

    mesh = plsc.VectorSubcoreMesh(core_axis_name="c", subcore_axis_name="s")

    @functools.partial(
        pl.kernel,
        mesh=mesh,
        out_type=jax.ShapeDtypeStruct((B, HIDDEN), jnp.float32),
        scratch_types=[
            pltpu.VMEM((nch, CH), jnp.int32),
            pltpu.VMEM((NBUF, CH, HIDDEN), jnp.float32),
            pltpu.SemaphoreType.DMA,
            pltpu.SemaphoreType.DMA,
            pltpu.SemaphoreType.DMA,
            pltpu.SemaphoreType.DMA,
            pltpu.SemaphoreType.DMA,
            pltpu.SemaphoreType.DMA,
            pltpu.SemaphoreType.DMA,
            pltpu.SemaphoreType.DMA,
        ],
    )
    def emb(ids_hbm, table_hbm, out_hbm, idx_v, rows_v,
            g0, g1, g2, g3, s0, s1, s2, s3):
        gsem = (g0, g1, g2, g3)
        ssem = (s0, s1, s2, s3)
        wid = lax.axis_index("s") * NC + lax.axis_index("c")
        pltpu.sync_copy(ids_hbm.at[wid], idx_v)
        base = wid * bpw

        def g_copy(c, b):
            return pltpu.make_async_copy(
                table_hbm.at[idx_v.at[c]], rows_v.at[b], gsem[b])

        def s_copy(c, b):
            return pltpu.make_async_copy(
                rows_v.at[b], out_hbm.at[pl.ds(base + c * CH, CH)], ssem[b])

        for b in range(LOOK):
            g_copy(b, b).start()

        def body(i, carry):
            for j in range(NBUF):
                c = i * NBUF + j

                @pl.when(c >= LOOK)
                def _():
                    s_copy(c - LOOK, (j + LOOK) % NBUF).wait()

                @pl.when(c + LOOK < nch)
                def _():
                    g_copy(c + LOOK, (j + LOOK) % NBUF).start()

                g_copy(c, j).wait()
                s_copy(c, j).start()

            return carry

        nloop = nch // NBUF
        lax.fori_loop(0, nloop, body, 0)

        for j in range(nch - nloop * NBUF):
            c = nloop * NBUF + j
            s_copy(c - LOOK, (j + LOOK) % NBUF).wait()
            g_copy(c, j).wait()
            s_copy(c, j).start()
        for c in range(nch - LOOK, nch):
            s_copy(c, c % NBUF).wait()

    return emb


def kernel(input_ids, embed_table):
    B = input_ids.size
    ids = input_ids.reshape(NW, B // (NW * CH), CH).astype(jnp.int32)
    out = _make_emb(B)(ids, embed_table)
    return out.reshape(-1, 1, HIDDEN)


# speedup vs baseline: 1.0100x; 1.0100x over previous
import functools

import jax
import jax.numpy as jnp
from jax import lax
from jax.experimental import pallas as pl
from jax.experimental.pallas import tpu as pltpu
from jax.experimental.pallas import tpu_sc as plsc

HIDDEN = 128
NC = 2
NS = 16
NW = NC * NS
CH = 128
NBUF = 7
LOOK = 3


@functools.lru_cache(maxsize=None)
def _make_emb(B):
    assert B % (NW * CH) == 0
    bpw = B // NW
    nch = bpw // CH
    assert nch > NBUF > LOOK

    mesh = plsc.VectorSubcoreMesh(core_axis_name="c", subcore_axis_name="s")

    @functools.partial(
        pl.kernel,
        mesh=mesh,
        out_type=jax.ShapeDtypeStruct((B, HIDDEN), jnp.float32),
        scratch_types=[
            pltpu.VMEM((nch, CH), jnp.int32),
            pltpu.VMEM((NBUF, CH, HIDDEN), jnp.float32),
        ] + [pltpu.SemaphoreType.DMA] * (2 * NBUF),
    )
    def emb(ids_hbm, table_hbm, out_hbm, idx_v, rows_v, *sems):
        gsem = sems[:NBUF]
        ssem = sems[NBUF:]
        wid = lax.axis_index("s") * NC + lax.axis_index("c")
        pltpu.sync_copy(ids_hbm.at[wid], idx_v)
        base = wid * bpw

        def g_copy(c, b):
            return pltpu.make_async_copy(
                table_hbm.at[idx_v.at[c]], rows_v.at[b], gsem[b])

        def s_copy(c, b):
            return pltpu.make_async_copy(
                rows_v.at[b], out_hbm.at[pl.ds(base + c * CH, CH)], ssem[b])

        for b in range(LOOK):
            g_copy(b, b).start()

        def slot(c, j, guarded):
            if guarded:
                @pl.when(c >= LOOK)
                def _():
                    s_copy(c - LOOK, (j - LOOK) % NBUF).wait()

                @pl.when(c + LOOK < nch)
                def _():
                    g_copy(c + LOOK, (j + LOOK) % NBUF).start()
            else:
                s_copy(c - LOOK, (j - LOOK) % NBUF).wait()
            g_copy(c, j).wait()
            s_copy(c, j).start()

        def body(i, carry):
            for j in range(NBUF):
                slot(i * NBUF + j, j, guarded=True)
            return carry

        nloop = nch // NBUF
        lax.fori_loop(0, nloop, body, 0)

        for j in range(nch - nloop * NBUF):
            slot(nloop * NBUF + j, j, guarded=False)
        for c in range(nch - LOOK, nch):
            s_copy(c, c % NBUF).wait()

    return emb


def kernel(input_ids, embed_table):
    B = input_ids.size
    ids = input_ids.reshape(NW, B // (NW * CH), CH).astype(jnp.int32)
    out = _make_emb(B)(ids, embed_table)
    return out.reshape(-1, 1, HIDDEN)


# speedup vs baseline: 1.0281x; 1.0179x over previous
import functools

import jax
import jax.numpy as jnp
from jax import lax
from jax.experimental import pallas as pl
from jax.experimental.pallas import tpu as pltpu
from jax.experimental.pallas import tpu_sc as plsc

HIDDEN = 128
NC = 2
NS = 16
NW = NC * NS
CH = 128
NBUF = 7
LOOK = 5


@functools.lru_cache(maxsize=None)
def _make_emb(B):
    assert B % (NW * CH) == 0
    bpw = B // NW
    nch = bpw // CH
    assert nch > NBUF > LOOK

    mesh = plsc.VectorSubcoreMesh(core_axis_name="c", subcore_axis_name="s")

    @functools.partial(
        pl.kernel,
        mesh=mesh,
        out_type=jax.ShapeDtypeStruct((B, HIDDEN), jnp.float32),
        scratch_types=[
            pltpu.VMEM((nch, CH), jnp.int32),
            pltpu.VMEM((NBUF, CH, HIDDEN), jnp.float32),
        ] + [pltpu.SemaphoreType.DMA] * (2 * NBUF),
    )
    def emb(ids_hbm, table_hbm, out_hbm, idx_v, rows_v, *sems):
        gsem = sems[:NBUF]
        ssem = sems[NBUF:]
        wid = lax.axis_index("s") * NC + lax.axis_index("c")
        pltpu.sync_copy(ids_hbm.at[wid], idx_v)
        base = wid * bpw

        def g_copy(c, b):
            return pltpu.make_async_copy(
                table_hbm.at[idx_v.at[c]], rows_v.at[b], gsem[b])

        def s_copy(c, b):
            return pltpu.make_async_copy(
                rows_v.at[b], out_hbm.at[pl.ds(base + c * CH, CH)], ssem[b])

        for b in range(LOOK):
            g_copy(b, b).start()

        def slot(c, j, guarded):
            if guarded:
                @pl.when(c >= LOOK)
                def _():
                    s_copy(c - LOOK, (j - LOOK) % NBUF).wait()

                @pl.when(c + LOOK < nch)
                def _():
                    g_copy(c + LOOK, (j + LOOK) % NBUF).start()
            else:
                s_copy(c - LOOK, (j - LOOK) % NBUF).wait()
            g_copy(c, j).wait()
            s_copy(c, j).start()

        def body(i, carry):
            for j in range(NBUF):
                slot(i * NBUF + j, j, guarded=True)
            return carry

        nloop = nch // NBUF
        lax.fori_loop(0, nloop, body, 0)

        for j in range(nch - nloop * NBUF):
            slot(nloop * NBUF + j, j, guarded=False)
        for c in range(nch - LOOK, nch):
            s_copy(c, c % NBUF).wait()

    return emb


def kernel(input_ids, embed_table):
    B = input_ids.size
    ids = input_ids.reshape(NW, B // (NW * CH), CH).astype(jnp.int32)
    out = _make_emb(B)(ids, embed_table)
    return out.reshape(-1, 1, HIDDEN)
